# Initial kernel scaffold; baseline (speedup 1.0000x reference)
import jax
import jax.numpy as jnp
from jax.experimental import pallas as pl


def kernel(x, W1, b1, W2, b2, W3, b3, edge_index):
    raise NotImplementedError("write your pallas kernel here")


# speedup vs baseline: 1.5828x; 1.5828x over previous
import functools

import jax
import jax.numpy as jnp
from jax import lax
from jax.experimental import pallas as pl
from jax.experimental.pallas import tpu as pltpu
from jax.experimental.pallas import tpu_sc as plsc

_EPS = 1e-5
_NC = 2
_NS = 16
_NW = _NC * _NS
_LANES = 16
_ND = 10240
_BD = 128
_RB = 1000


def _sc_mesh():
    return plsc.VectorSubcoreMesh(core_axis_name="c", subcore_axis_name="s")


def _make_degree_kernel(e, e_pad, nb_per_tile):
    nr = _ND // _LANES

    @functools.partial(
        pl.kernel,
        mesh=_sc_mesh(),
        out_type=jax.ShapeDtypeStruct((_NW, nr, _LANES), jnp.float32),
        scratch_types=[
            pltpu.VMEM((_BD,), jnp.int32),
            pltpu.VMEM((nr, _LANES), jnp.float32),
        ],
    )
    def degree_kernel(rows_hbm, zeros_hbm, out_hbm, idx_v, hist_v):
        c = lax.axis_index("c")
        s = lax.axis_index("s")
        w = s * _NC + c
        pltpu.sync_copy(zeros_hbm, hist_v)
        lanes = lax.iota(jnp.int32, _LANES)
        one = jnp.full((_LANES,), 1.0, jnp.float32)
        zero = jnp.zeros((_LANES,), jnp.float32)
        nb_all = (e + _BD - 1) // _BD
        nb_t = jnp.maximum((nb_all - w + _NW - 1) // _NW, 0)

        def _batch(m, carry):
            base = (w + m * _NW) * _BD
            pltpu.sync_copy(rows_hbm.at[pl.ds(base, _BD)], idx_v)
            for q in range(_BD // _LANES):
                pos = base + q * _LANES + lanes
                v = idx_v[pl.ds(q * _LANES, _LANES)]
                vm = jnp.where(pos < e, v, _ND - 1)
                for j in range(_LANES):
                    r = vm[j]
                    row = lax.shift_right_logical(r, 4)
                    lane = lax.bitwise_and(r, 15)
                    hist_v[row, :] += jnp.where(lanes == lane, one, zero)
            return carry

        lax.fori_loop(0, nb_t, _batch, 0)
        pltpu.sync_copy(hist_v, out_hbm.at[w])

    return degree_kernel


def _make_propagate_kernel(n, d):
    ct = 81920 // d
    bsz = 32768 // d
    sweeps = _ND // (_NW * ct)
    assert _ND == sweeps * _NW * ct and ct % 8 == 0 and bsz % _LANES == 0

    @functools.partial(
        pl.kernel,
        mesh=_sc_mesh(),
        out_type=jax.ShapeDtypeStruct((_ND, d), jnp.float32),
        scratch_types=[
            pltpu.VMEM((bsz,), jnp.int32),
            pltpu.VMEM((bsz,), jnp.int32),
            pltpu.VMEM((bsz, d), jnp.float32),
            pltpu.VMEM((1, _LANES), jnp.int32),
            pltpu.VMEM((ct + 1, d), jnp.float32),
            pltpu.SemaphoreType.DMA,
        ],
    )
    def propagate_kernel(tsc_hbm, row_hbm, col_hbm, bnd_hbm, zrows_hbm,
                         out_hbm, ridx_v, cidx_v, msg_v, bvec_v, acc_v, sem):
        c = lax.axis_index("c")
        s = lax.axis_index("s")
        w = s * _NC + c

        def _sweep(p, carry0):
            chunk = p * _NW + w
            base_rows = chunk * ct
            pltpu.sync_copy(zrows_hbm, acc_v.at[pl.ds(0, ct)])
            pltpu.sync_copy(bnd_hbm.at[chunk], bvec_v)
            bv = bvec_v[0, pl.ds(0, _LANES)]
            s_k = bv[0]
            e_k = bv[1]
            s_k8 = (s_k // 8) * 8
            nb = (e_k - s_k8 + bsz - 1) // bsz

            def _batch(m, carry):
                base = s_k8 + m * bsz
                pltpu.sync_copy(row_hbm.at[pl.ds(base, bsz)], ridx_v)
                pltpu.sync_copy(col_hbm.at[pl.ds(base, bsz)], cidx_v)
                for q in range(bsz // _LANES):
                    pos = base + q * _LANES + lax.iota(jnp.int32, _LANES)
                    ok = (pos >= s_k) & (pos < e_k)
                    rv = ridx_v[pl.ds(q * _LANES, _LANES)]
                    cv = cidx_v[pl.ds(q * _LANES, _LANES)]
                    ridx_v[pl.ds(q * _LANES, _LANES)] = jnp.where(ok, rv, 0)
                    cidx_v[pl.ds(q * _LANES, _LANES)] = jnp.where(
                        ok, cv - base_rows, ct)
                pltpu.async_copy(tsc_hbm.at[ridx_v], msg_v, sem).wait()
                for g in range(bsz // _LANES):
                    cvec = cidx_v[pl.ds(g * _LANES, _LANES)]
                    for j in range(_LANES):
                        r = cvec[j]
                        for q in range(d // _LANES):
                            acc_v[r, pl.ds(q * _LANES, _LANES)] += msg_v[
                                g * _LANES + j, pl.ds(q * _LANES, _LANES)]
                return carry

            lax.fori_loop(0, nb, _batch, 0)
            pltpu.sync_copy(acc_v.at[pl.ds(0, ct)],
                            out_hbm.at[pl.ds(base_rows, ct)])
            return carry0

        lax.fori_loop(0, sweeps, _sweep, 0)

    return propagate_kernel


def _bin_weights_body(w1_ref, w2_ref, w3_ref, s1_ref, s2_ref, s3_ref, m1_ref,
                      m2_ref, m3_ref):
    for wr, sr, mr in ((w1_ref, s1_ref, m1_ref), (w2_ref, s2_ref, m2_ref),
                       (w3_ref, s3_ref, m3_ref)):
        w = wr[...]
        mr[...] = jnp.mean(jnp.abs(w), axis=1)[None, :]
        sr[...] = jnp.sign(w).astype(jnp.bfloat16)


def _binarize_weights(W1, W2, W3):
    h, din = W1.shape
    out_d = W3.shape[0]
    return pl.pallas_call(
        _bin_weights_body,
        out_shape=[
            jax.ShapeDtypeStruct((h, din), jnp.bfloat16),
            jax.ShapeDtypeStruct((h, h), jnp.bfloat16),
            jax.ShapeDtypeStruct((out_d, h), jnp.bfloat16),
            jax.ShapeDtypeStruct((1, h), jnp.float32),
            jax.ShapeDtypeStruct((1, h), jnp.float32),
            jax.ShapeDtypeStruct((1, out_d), jnp.float32),
        ],
    )(W1, W2, W3)


def _dinv_of(deg_block):
    deg = jnp.sum(deg_block, axis=1) + 1.0
    return lax.rsqrt(deg)[:, None]


def _binlinear(h, ws_ref, wm_ref, b_ref, dinv):
    alpha = jnp.mean(jnp.abs(h), axis=1, keepdims=True)
    hs = jnp.sign(h).astype(jnp.bfloat16)
    acc = lax.dot_general(hs, ws_ref[...], (((1,), (1,)), ((), ())),
                          preferred_element_type=jnp.float32)
    return dinv * (acc * (alpha * wm_ref[...]) + b_ref[...])


def _d_first_body(x_ref, deg_ref, ws_ref, wm_ref, b_ref, out_ref):
    inv0 = (1.0 + _EPS) ** -0.5
    h = x_ref[...] * inv0
    out_ref[...] = _binlinear(h, ws_ref, wm_ref, b_ref, _dinv_of(deg_ref[...]))


def _d_mid_body(agg_ref, tsc_ref, deg_ref, ws_ref, wm_ref, b_ref, out_ref):
    dinv = _dinv_of(deg_ref[...])
    h = dinv * (agg_ref[...] + tsc_ref[...])
    out_ref[...] = _binlinear(h, ws_ref, wm_ref, b_ref, dinv)


def _d_final_body(agg_ref, tsc_ref, deg_ref, out_ref):
    dinv = _dinv_of(deg_ref[...])
    h = dinv * (agg_ref[...] + tsc_ref[...])
    mx = jnp.max(h, axis=1, keepdims=True)
    lse = jnp.log(jnp.sum(jnp.exp(h - mx), axis=1, keepdims=True)) + mx
    out_ref[...] = h - lse


def _row_spec(din):
    return pl.BlockSpec((_RB, din), lambda i: (i, 0))


def _deg_spec(n1):
    return pl.BlockSpec((_RB, _NW), lambda i: (i, 0))


def _full_spec(shape):
    return pl.BlockSpec(shape, lambda i: tuple(0 for _ in shape))


def _dense_first(x, deg16, ws, wm, b2d):
    n, din = x.shape
    dout = ws.shape[0]
    return pl.pallas_call(
        _d_first_body,
        grid=(n // _RB,),
        in_specs=[
            _row_spec(din),
            _deg_spec(n + 1),
            _full_spec(ws.shape),
            _full_spec(wm.shape),
            _full_spec(b2d.shape),
        ],
        out_specs=_row_spec(dout),
        out_shape=jax.ShapeDtypeStruct((n, dout), jnp.float32),
    )(x, deg16, ws, wm, b2d)


def _dense_mid(agg, tsc, deg16, ws, wm, b2d):
    n, din = tsc.shape
    dout = ws.shape[0]
    return pl.pallas_call(
        _d_mid_body,
        grid=(n // _RB,),
        in_specs=[
            _row_spec(din),
            _row_spec(din),
            _deg_spec(n + 1),
            _full_spec(ws.shape),
            _full_spec(wm.shape),
            _full_spec(b2d.shape),
        ],
        out_specs=_row_spec(dout),
        out_shape=jax.ShapeDtypeStruct((n, dout), jnp.float32),
    )(agg, tsc, deg16, ws, wm, b2d)


def _dense_final(agg, tsc, deg16):
    n, d = tsc.shape
    return pl.pallas_call(
        _d_final_body,
        grid=(n // _RB,),
        in_specs=[_row_spec(d), _row_spec(d), _deg_spec(n + 1)],
        out_specs=_row_spec(d),
        out_shape=jax.ShapeDtypeStruct((n, d), jnp.float32),
    )(agg, tsc, deg16)


def kernel(x, W1, b1, W2, b2, W3, b3, edge_index):
    n, _ = x.shape
    h1 = W1.shape[0]
    out_d = W3.shape[0]
    e = edge_index.shape[1]

    row = edge_index[0]
    col = edge_index[1]
    order = jnp.argsort(col)
    col_s = jnp.take(col, order)
    row_s = jnp.take(row, order)
    nb_deg = -(-e // _BD)
    e_pad = (-(-nb_deg // _NW)) * _NW * _BD
    pad = jnp.zeros((e_pad - e,), jnp.int32)
    row_p = jnp.concatenate([row_s, pad])
    col_p = jnp.concatenate([col_s, pad])

    def _bounds_for(d):
        ct = 81920 // d
        nchunks = _ND // ct
        starts = jnp.searchsorted(
            col_s, jnp.arange(0, _ND + ct, ct, dtype=jnp.int32)).astype(
                jnp.int32)
        bnd = jnp.stack([starts[:-1], starts[1:]], axis=1)
        bnd = jnp.concatenate(
            [bnd, jnp.zeros((nchunks, 14), jnp.int32)], axis=1)
        return bnd.reshape(nchunks, 1, 16)

    zeros_deg = jnp.zeros((_ND // _LANES, _LANES), jnp.float32)
    zrows_h = jnp.zeros((81920 // h1, h1), jnp.float32)
    zrows_o = jnp.zeros((81920 // out_d, out_d), jnp.float32)

    deg_parts = _make_degree_kernel(e, e_pad, nb_deg)(row_p, zeros_deg)
    deg_flat = deg_parts.reshape(_NW, _ND).T
    ws1, ws2, ws3, m1, m2, m3 = _binarize_weights(W1, W2, W3)
    b1r, b2r, b3r = b1[None, :], b2[None, :], b3[None, :]

    prop_h = _make_propagate_kernel(n, h1)
    prop_o = _make_propagate_kernel(n, out_d)
    bnd_h = _bounds_for(h1)
    bnd_o = _bounds_for(out_d)

    t1 = _dense_first(x, deg_flat, ws1, m1, b1r)
    a1 = prop_h(t1, row_p, col_p, bnd_h, zrows_h)
    t2 = _dense_mid(a1, t1, deg_flat, ws2, m2, b2r)
    a2 = prop_h(t2, row_p, col_p, bnd_h, zrows_h)
    t3 = _dense_mid(a2, t2, deg_flat, ws3, m3, b3r)
    a3 = prop_o(t3, row_p, col_p, bnd_o, zrows_o)
    return _dense_final(a3, t3, deg_flat)


# speedup vs baseline: 2.1107x; 1.3335x over previous
import functools

import jax
import jax.numpy as jnp
from jax import lax
from jax.experimental import pallas as pl
from jax.experimental.pallas import tpu as pltpu
from jax.experimental.pallas import tpu_sc as plsc

_EPS = 1e-5
_NC = 2
_NS = 16
_NW = _NC * _NS
_LANES = 16
_ND = 10240
_BD = 128
_RB = 1000


def _sc_mesh():
    return plsc.VectorSubcoreMesh(core_axis_name="c", subcore_axis_name="s")


def _make_degree_kernel(e, e_pad, nb_per_tile):
    nr = _ND // _LANES

    @functools.partial(
        pl.kernel,
        mesh=_sc_mesh(),
        out_type=jax.ShapeDtypeStruct((_NW, nr, _LANES), jnp.float32),
        scratch_types=[
            pltpu.VMEM((_BD,), jnp.int32),
            pltpu.VMEM((nr, _LANES), jnp.float32),
        ],
    )
    def degree_kernel(rows_hbm, zeros_hbm, out_hbm, idx_v, hist_v):
        c = lax.axis_index("c")
        s = lax.axis_index("s")
        w = s * _NC + c
        pltpu.sync_copy(zeros_hbm, hist_v)
        lanes = lax.iota(jnp.int32, _LANES)
        one = jnp.full((_LANES,), 1.0, jnp.float32)
        zero = jnp.zeros((_LANES,), jnp.float32)
        nb_all = (e + _BD - 1) // _BD
        nb_t = jnp.maximum((nb_all - w + _NW - 1) // _NW, 0)

        def _batch(m, carry):
            base = (w + m * _NW) * _BD
            pltpu.sync_copy(rows_hbm.at[pl.ds(base, _BD)], idx_v)
            for q in range(_BD // _LANES):
                pos = base + q * _LANES + lanes
                v = idx_v[pl.ds(q * _LANES, _LANES)]
                vm = jnp.where(pos < e, v, _ND - 1)
                for j in range(_LANES):
                    r = vm[j]
                    row = lax.shift_right_logical(r, 4)
                    lane = lax.bitwise_and(r, 15)
                    hist_v[row, :] += jnp.where(lanes == lane, one, zero)
            return carry

        lax.fori_loop(0, nb_t, _batch, 0)
        pltpu.sync_copy(hist_v, out_hbm.at[w])

    return degree_kernel


def _make_propagate_kernel(n, d):
    ct = 81920 // d
    bsz = 32768 // d
    sweeps = _ND // (_NW * ct)
    assert _ND == sweeps * _NW * ct and ct % 8 == 0 and bsz % _LANES == 0

    @functools.partial(
        pl.kernel,
        mesh=_sc_mesh(),
        out_type=jax.ShapeDtypeStruct((_ND, d), jnp.float32),
        scratch_types=[
            pltpu.VMEM((bsz,), jnp.int32),
            pltpu.VMEM((bsz,), jnp.int32),
            pltpu.VMEM((bsz, d), jnp.float32),
            pltpu.VMEM((1, _LANES), jnp.int32),
            pltpu.VMEM((ct + 1, d), jnp.float32),
            pltpu.SemaphoreType.DMA,
        ],
    )
    def propagate_kernel(tsc_hbm, row_hbm, col_hbm, bnd_hbm, zrows_hbm,
                         out_hbm, ridx_v, cidx_v, msg_v, bvec_v, acc_v, sem):
        c = lax.axis_index("c")
        s = lax.axis_index("s")
        w = s * _NC + c

        def _sweep(p, carry0):
            chunk = p * _NW + w
            base_rows = chunk * ct
            pltpu.sync_copy(zrows_hbm, acc_v.at[pl.ds(0, ct)])
            pltpu.sync_copy(bnd_hbm.at[chunk], bvec_v)
            bv = bvec_v[0, pl.ds(0, _LANES)]
            s_k = bv[0]
            e_k = bv[1]
            s_k8 = (s_k // 8) * 8
            nb = (e_k - s_k8 + bsz - 1) // bsz

            def _batch(m, carry):
                base = s_k8 + m * bsz
                pltpu.sync_copy(row_hbm.at[pl.ds(base, bsz)], ridx_v)
                pltpu.sync_copy(col_hbm.at[pl.ds(base, bsz)], cidx_v)
                for q in range(bsz // _LANES):
                    pos = base + q * _LANES + lax.iota(jnp.int32, _LANES)
                    ok = (pos >= s_k) & (pos < e_k)
                    rv = ridx_v[pl.ds(q * _LANES, _LANES)]
                    cv = cidx_v[pl.ds(q * _LANES, _LANES)]
                    ridx_v[pl.ds(q * _LANES, _LANES)] = jnp.where(ok, rv, 0)
                    cidx_v[pl.ds(q * _LANES, _LANES)] = jnp.where(
                        ok, cv - base_rows, ct)
                pltpu.async_copy(tsc_hbm.at[ridx_v], msg_v, sem).wait()
                for g in range(bsz // _LANES):
                    cvec = cidx_v[pl.ds(g * _LANES, _LANES)]
                    for j in range(_LANES):
                        r = cvec[j]
                        for q in range(d // _LANES):
                            plsc.addupdate(
                                acc_v.at[r, pl.ds(q * _LANES, _LANES)],
                                msg_v[g * _LANES + j,
                                      pl.ds(q * _LANES, _LANES)])
                return carry

            lax.fori_loop(0, nb, _batch, 0)
            pltpu.sync_copy(acc_v.at[pl.ds(0, ct)],
                            out_hbm.at[pl.ds(base_rows, ct)])
            return carry0

        lax.fori_loop(0, sweeps, _sweep, 0)

    return propagate_kernel


def _bin_weights_body(w1_ref, w2_ref, w3_ref, s1_ref, s2_ref, s3_ref, m1_ref,
                      m2_ref, m3_ref):
    for wr, sr, mr in ((w1_ref, s1_ref, m1_ref), (w2_ref, s2_ref, m2_ref),
                       (w3_ref, s3_ref, m3_ref)):
        w = wr[...]
        mr[...] = jnp.mean(jnp.abs(w), axis=1)[None, :]
        sr[...] = jnp.sign(w).astype(jnp.bfloat16)


def _binarize_weights(W1, W2, W3):
    h, din = W1.shape
    out_d = W3.shape[0]
    return pl.pallas_call(
        _bin_weights_body,
        out_shape=[
            jax.ShapeDtypeStruct((h, din), jnp.bfloat16),
            jax.ShapeDtypeStruct((h, h), jnp.bfloat16),
            jax.ShapeDtypeStruct((out_d, h), jnp.bfloat16),
            jax.ShapeDtypeStruct((1, h), jnp.float32),
            jax.ShapeDtypeStruct((1, h), jnp.float32),
            jax.ShapeDtypeStruct((1, out_d), jnp.float32),
        ],
    )(W1, W2, W3)


def _dinv_of(deg_block):
    deg = jnp.sum(deg_block, axis=1) + 1.0
    return lax.rsqrt(deg)[:, None]


def _binlinear(h, ws_ref, wm_ref, b_ref, dinv):
    alpha = jnp.mean(jnp.abs(h), axis=1, keepdims=True)
    hs = jnp.sign(h).astype(jnp.bfloat16)
    acc = lax.dot_general(hs, ws_ref[...], (((1,), (1,)), ((), ())),
                          preferred_element_type=jnp.float32)
    return dinv * (acc * (alpha * wm_ref[...]) + b_ref[...])


def _d_first_body(x_ref, deg_ref, ws_ref, wm_ref, b_ref, out_ref):
    inv0 = (1.0 + _EPS) ** -0.5
    h = x_ref[...] * inv0
    out_ref[...] = _binlinear(h, ws_ref, wm_ref, b_ref, _dinv_of(deg_ref[...]))


def _d_mid_body(agg_ref, tsc_ref, deg_ref, ws_ref, wm_ref, b_ref, out_ref):
    dinv = _dinv_of(deg_ref[...])
    h = dinv * (agg_ref[...] + tsc_ref[...])
    out_ref[...] = _binlinear(h, ws_ref, wm_ref, b_ref, dinv)


def _d_final_body(agg_ref, tsc_ref, deg_ref, out_ref):
    dinv = _dinv_of(deg_ref[...])
    h = dinv * (agg_ref[...] + tsc_ref[...])
    mx = jnp.max(h, axis=1, keepdims=True)
    lse = jnp.log(jnp.sum(jnp.exp(h - mx), axis=1, keepdims=True)) + mx
    out_ref[...] = h - lse


def _row_spec(din):
    return pl.BlockSpec((_RB, din), lambda i: (i, 0))


def _deg_spec(n1):
    return pl.BlockSpec((_RB, _NW), lambda i: (i, 0))


def _full_spec(shape):
    return pl.BlockSpec(shape, lambda i: tuple(0 for _ in shape))


def _dense_first(x, deg16, ws, wm, b2d):
    n, din = x.shape
    dout = ws.shape[0]
    return pl.pallas_call(
        _d_first_body,
        grid=(n // _RB,),
        in_specs=[
            _row_spec(din),
            _deg_spec(n + 1),
            _full_spec(ws.shape),
            _full_spec(wm.shape),
            _full_spec(b2d.shape),
        ],
        out_specs=_row_spec(dout),
        out_shape=jax.ShapeDtypeStruct((n, dout), jnp.float32),
    )(x, deg16, ws, wm, b2d)


def _dense_mid(agg, tsc, deg16, ws, wm, b2d):
    n, din = tsc.shape
    dout = ws.shape[0]
    return pl.pallas_call(
        _d_mid_body,
        grid=(n // _RB,),
        in_specs=[
            _row_spec(din),
            _row_spec(din),
            _deg_spec(n + 1),
            _full_spec(ws.shape),
            _full_spec(wm.shape),
            _full_spec(b2d.shape),
        ],
        out_specs=_row_spec(dout),
        out_shape=jax.ShapeDtypeStruct((n, dout), jnp.float32),
    )(agg, tsc, deg16, ws, wm, b2d)


def _dense_final(agg, tsc, deg16):
    n, d = tsc.shape
    return pl.pallas_call(
        _d_final_body,
        grid=(n // _RB,),
        in_specs=[_row_spec(d), _row_spec(d), _deg_spec(n + 1)],
        out_specs=_row_spec(d),
        out_shape=jax.ShapeDtypeStruct((n, d), jnp.float32),
    )(agg, tsc, deg16)


def kernel(x, W1, b1, W2, b2, W3, b3, edge_index):
    n, _ = x.shape
    h1 = W1.shape[0]
    out_d = W3.shape[0]
    e = edge_index.shape[1]

    row = edge_index[0]
    col = edge_index[1]
    order = jnp.argsort(col)
    col_s = jnp.take(col, order)
    row_s = jnp.take(row, order)
    nb_deg = -(-e // _BD)
    e_pad = (-(-nb_deg // _NW)) * _NW * _BD
    pad = jnp.zeros((e_pad - e,), jnp.int32)
    row_p = jnp.concatenate([row_s, pad])
    col_p = jnp.concatenate([col_s, pad])

    def _bounds_for(d):
        ct = 81920 // d
        nchunks = _ND // ct
        starts = jnp.searchsorted(
            col_s, jnp.arange(0, _ND + ct, ct, dtype=jnp.int32)).astype(
                jnp.int32)
        bnd = jnp.stack([starts[:-1], starts[1:]], axis=1)
        bnd = jnp.concatenate(
            [bnd, jnp.zeros((nchunks, 14), jnp.int32)], axis=1)
        return bnd.reshape(nchunks, 1, 16)

    zeros_deg = jnp.zeros((_ND // _LANES, _LANES), jnp.float32)
    zrows_h = jnp.zeros((81920 // h1, h1), jnp.float32)
    zrows_o = jnp.zeros((81920 // out_d, out_d), jnp.float32)

    deg_parts = _make_degree_kernel(e, e_pad, nb_deg)(row_p, zeros_deg)
    deg_flat = deg_parts.reshape(_NW, _ND).T
    ws1, ws2, ws3, m1, m2, m3 = _binarize_weights(W1, W2, W3)
    b1r, b2r, b3r = b1[None, :], b2[None, :], b3[None, :]

    prop_h = _make_propagate_kernel(n, h1)
    prop_o = _make_propagate_kernel(n, out_d)
    bnd_h = _bounds_for(h1)
    bnd_o = _bounds_for(out_d)

    t1 = _dense_first(x, deg_flat, ws1, m1, b1r)
    a1 = prop_h(t1, row_p, col_p, bnd_h, zrows_h)
    t2 = _dense_mid(a1, t1, deg_flat, ws2, m2, b2r)
    a2 = prop_h(t2, row_p, col_p, bnd_h, zrows_h)
    t3 = _dense_mid(a2, t2, deg_flat, ws3, m3, b3r)
    a3 = prop_o(t3, row_p, col_p, bnd_o, zrows_o)
    return _dense_final(a3, t3, deg_flat)


# speedup vs baseline: 2.5056x; 1.1871x over previous
import functools

import jax
import jax.numpy as jnp
from jax import lax
from jax.experimental import pallas as pl
from jax.experimental.pallas import tpu as pltpu
from jax.experimental.pallas import tpu_sc as plsc

_EPS = 1e-5
_NC = 2
_NS = 16
_NW = _NC * _NS
_LANES = 16
_ND = 10240
_BD = 128
_RB = 1000


def _sc_mesh():
    return plsc.VectorSubcoreMesh(core_axis_name="c", subcore_axis_name="s")


def _make_degree_kernel(e, e_pad, nb_per_tile):
    nr = _ND // _LANES

    @functools.partial(
        pl.kernel,
        mesh=_sc_mesh(),
        out_type=jax.ShapeDtypeStruct((_NW, nr, _LANES), jnp.float32),
        scratch_types=[
            pltpu.VMEM((_BD,), jnp.int32),
            pltpu.VMEM((nr, _LANES), jnp.float32),
        ],
    )
    def degree_kernel(rows_hbm, zeros_hbm, out_hbm, idx_v, hist_v):
        c = lax.axis_index("c")
        s = lax.axis_index("s")
        w = s * _NC + c
        pltpu.sync_copy(zeros_hbm, hist_v)
        lanes = lax.iota(jnp.int32, _LANES)
        one = jnp.full((_LANES,), 1.0, jnp.float32)
        zero = jnp.zeros((_LANES,), jnp.float32)
        nb_all = (e + _BD - 1) // _BD
        nb_t = jnp.maximum((nb_all - w + _NW - 1) // _NW, 0)

        def _batch(m, carry):
            base = (w + m * _NW) * _BD
            pltpu.sync_copy(rows_hbm.at[pl.ds(base, _BD)], idx_v)
            for q in range(_BD // _LANES):
                pos = base + q * _LANES + lanes
                v = idx_v[pl.ds(q * _LANES, _LANES)]
                vm = jnp.where(pos < e, v, _ND - 1)
                for j in range(_LANES):
                    r = vm[j]
                    row = lax.shift_right_logical(r, 4)
                    lane = lax.bitwise_and(r, 15)
                    hist_v[row, :] += jnp.where(lanes == lane, one, zero)
            return carry

        lax.fori_loop(0, nb_t, _batch, 0)
        pltpu.sync_copy(hist_v, out_hbm.at[w])

    return degree_kernel


def _make_propagate_kernel(n, d):
    ct = 81920 // d
    bsz = 32768 // d
    sweeps = _ND // (_NW * ct)
    assert _ND == sweeps * _NW * ct and ct % 8 == 0 and bsz % _LANES == 0

    @functools.partial(
        pl.kernel,
        mesh=_sc_mesh(),
        out_type=jax.ShapeDtypeStruct((_ND, d), jnp.float32),
        scratch_types=[
            pltpu.VMEM((bsz,), jnp.int32),
            pltpu.VMEM((bsz,), jnp.int32),
            pltpu.VMEM((bsz, d), jnp.float32),
            pltpu.VMEM((1, _LANES), jnp.int32),
            pltpu.VMEM((ct + 1, d), jnp.float32),
            pltpu.SemaphoreType.DMA,
        ],
    )
    def propagate_kernel(tsc_hbm, row_hbm, col_hbm, bnd_hbm, zrows_hbm,
                         out_hbm, ridx_v, cidx_v, msg_v, bvec_v, acc_v, sem):
        c = lax.axis_index("c")
        s = lax.axis_index("s")
        w = s * _NC + c

        def _sweep(p, carry0):
            chunk = p * _NW + w
            base_rows = chunk * ct
            pltpu.sync_copy(zrows_hbm, acc_v.at[pl.ds(0, ct)])
            pltpu.sync_copy(bnd_hbm.at[chunk], bvec_v)
            bv = bvec_v[0, pl.ds(0, _LANES)]
            s_k = bv[0]
            e_k = bv[1]
            s_k8 = (s_k // 8) * 8
            nb = (e_k - s_k8 + bsz - 1) // bsz

            def _batch(m, carry):
                base = s_k8 + m * bsz
                pltpu.sync_copy(row_hbm.at[pl.ds(base, bsz)], ridx_v)
                pltpu.sync_copy(col_hbm.at[pl.ds(base, bsz)], cidx_v)
                for q in range(bsz // _LANES):
                    pos = base + q * _LANES + lax.iota(jnp.int32, _LANES)
                    ok = (pos >= s_k) & (pos < e_k)
                    rv = ridx_v[pl.ds(q * _LANES, _LANES)]
                    cv = cidx_v[pl.ds(q * _LANES, _LANES)]
                    ridx_v[pl.ds(q * _LANES, _LANES)] = jnp.where(ok, rv, 0)
                    cidx_v[pl.ds(q * _LANES, _LANES)] = jnp.where(
                        ok, cv - base_rows, ct)
                pltpu.async_copy(tsc_hbm.at[ridx_v], msg_v, sem).wait()
                for g in range(bsz // _LANES):
                    cvec = cidx_v[pl.ds(g * _LANES, _LANES)]
                    rs = [cvec[j] for j in range(_LANES)]

                    def _qbody(q, carry2, g=g, rs=rs):
                        for j in range(_LANES):
                            plsc.addupdate(
                                acc_v.at[rs[j], pl.ds(q * _LANES, _LANES)],
                                msg_v[g * _LANES + j,
                                      pl.ds(q * _LANES, _LANES)])
                        return carry2

                    lax.fori_loop(0, d // _LANES, _qbody, 0)
                return carry

            lax.fori_loop(0, nb, _batch, 0)
            pltpu.sync_copy(acc_v.at[pl.ds(0, ct)],
                            out_hbm.at[pl.ds(base_rows, ct)])
            return carry0

        lax.fori_loop(0, sweeps, _sweep, 0)

    return propagate_kernel


def _bin_weights_body(w1_ref, w2_ref, w3_ref, s1_ref, s2_ref, s3_ref, m1_ref,
                      m2_ref, m3_ref):
    for wr, sr, mr in ((w1_ref, s1_ref, m1_ref), (w2_ref, s2_ref, m2_ref),
                       (w3_ref, s3_ref, m3_ref)):
        w = wr[...]
        mr[...] = jnp.mean(jnp.abs(w), axis=1)[None, :]
        sr[...] = jnp.sign(w).astype(jnp.bfloat16)


def _binarize_weights(W1, W2, W3):
    h, din = W1.shape
    out_d = W3.shape[0]
    return pl.pallas_call(
        _bin_weights_body,
        out_shape=[
            jax.ShapeDtypeStruct((h, din), jnp.bfloat16),
            jax.ShapeDtypeStruct((h, h), jnp.bfloat16),
            jax.ShapeDtypeStruct((out_d, h), jnp.bfloat16),
            jax.ShapeDtypeStruct((1, h), jnp.float32),
            jax.ShapeDtypeStruct((1, h), jnp.float32),
            jax.ShapeDtypeStruct((1, out_d), jnp.float32),
        ],
    )(W1, W2, W3)


def _dinv_of(deg_block):
    deg = jnp.sum(deg_block, axis=1) + 1.0
    return lax.rsqrt(deg)[:, None]


def _binlinear(h, ws_ref, wm_ref, b_ref, dinv):
    alpha = jnp.mean(jnp.abs(h), axis=1, keepdims=True)
    hs = jnp.sign(h).astype(jnp.bfloat16)
    acc = lax.dot_general(hs, ws_ref[...], (((1,), (1,)), ((), ())),
                          preferred_element_type=jnp.float32)
    return dinv * (acc * (alpha * wm_ref[...]) + b_ref[...])


def _d_first_body(x_ref, deg_ref, ws_ref, wm_ref, b_ref, out_ref):
    inv0 = (1.0 + _EPS) ** -0.5
    h = x_ref[...] * inv0
    out_ref[...] = _binlinear(h, ws_ref, wm_ref, b_ref, _dinv_of(deg_ref[...]))


def _d_mid_body(agg_ref, tsc_ref, deg_ref, ws_ref, wm_ref, b_ref, out_ref):
    dinv = _dinv_of(deg_ref[...])
    h = dinv * (agg_ref[...] + tsc_ref[...])
    out_ref[...] = _binlinear(h, ws_ref, wm_ref, b_ref, dinv)


def _d_final_body(agg_ref, tsc_ref, deg_ref, out_ref):
    dinv = _dinv_of(deg_ref[...])
    h = dinv * (agg_ref[...] + tsc_ref[...])
    mx = jnp.max(h, axis=1, keepdims=True)
    lse = jnp.log(jnp.sum(jnp.exp(h - mx), axis=1, keepdims=True)) + mx
    out_ref[...] = h - lse


def _row_spec(din):
    return pl.BlockSpec((_RB, din), lambda i: (i, 0))


def _deg_spec(n1):
    return pl.BlockSpec((_RB, _NW), lambda i: (i, 0))


def _full_spec(shape):
    return pl.BlockSpec(shape, lambda i: tuple(0 for _ in shape))


def _dense_first(x, deg16, ws, wm, b2d):
    n, din = x.shape
    dout = ws.shape[0]
    return pl.pallas_call(
        _d_first_body,
        grid=(n // _RB,),
        in_specs=[
            _row_spec(din),
            _deg_spec(n + 1),
            _full_spec(ws.shape),
            _full_spec(wm.shape),
            _full_spec(b2d.shape),
        ],
        out_specs=_row_spec(dout),
        out_shape=jax.ShapeDtypeStruct((n, dout), jnp.float32),
    )(x, deg16, ws, wm, b2d)


def _dense_mid(agg, tsc, deg16, ws, wm, b2d):
    n, din = tsc.shape
    dout = ws.shape[0]
    return pl.pallas_call(
        _d_mid_body,
        grid=(n // _RB,),
        in_specs=[
            _row_spec(din),
            _row_spec(din),
            _deg_spec(n + 1),
            _full_spec(ws.shape),
            _full_spec(wm.shape),
            _full_spec(b2d.shape),
        ],
        out_specs=_row_spec(dout),
        out_shape=jax.ShapeDtypeStruct((n, dout), jnp.float32),
    )(agg, tsc, deg16, ws, wm, b2d)


def _dense_final(agg, tsc, deg16):
    n, d = tsc.shape
    return pl.pallas_call(
        _d_final_body,
        grid=(n // _RB,),
        in_specs=[_row_spec(d), _row_spec(d), _deg_spec(n + 1)],
        out_specs=_row_spec(d),
        out_shape=jax.ShapeDtypeStruct((n, d), jnp.float32),
    )(agg, tsc, deg16)


def kernel(x, W1, b1, W2, b2, W3, b3, edge_index):
    n, _ = x.shape
    h1 = W1.shape[0]
    out_d = W3.shape[0]
    e = edge_index.shape[1]

    row = edge_index[0]
    col = edge_index[1]
    order = jnp.argsort(col)
    col_s = jnp.take(col, order)
    row_s = jnp.take(row, order)
    nb_deg = -(-e // _BD)
    e_pad = (-(-nb_deg // _NW)) * _NW * _BD
    pad = jnp.zeros((e_pad - e,), jnp.int32)
    row_p = jnp.concatenate([row_s, pad])
    col_p = jnp.concatenate([col_s, pad])

    def _bounds_for(d):
        ct = 81920 // d
        nchunks = _ND // ct
        starts = jnp.searchsorted(
            col_s, jnp.arange(0, _ND + ct, ct, dtype=jnp.int32)).astype(
                jnp.int32)
        bnd = jnp.stack([starts[:-1], starts[1:]], axis=1)
        bnd = jnp.concatenate(
            [bnd, jnp.zeros((nchunks, 14), jnp.int32)], axis=1)
        return bnd.reshape(nchunks, 1, 16)

    zeros_deg = jnp.zeros((_ND // _LANES, _LANES), jnp.float32)
    zrows_h = jnp.zeros((81920 // h1, h1), jnp.float32)
    zrows_o = jnp.zeros((81920 // out_d, out_d), jnp.float32)

    deg_parts = _make_degree_kernel(e, e_pad, nb_deg)(row_p, zeros_deg)
    deg_flat = deg_parts.reshape(_NW, _ND).T
    ws1, ws2, ws3, m1, m2, m3 = _binarize_weights(W1, W2, W3)
    b1r, b2r, b3r = b1[None, :], b2[None, :], b3[None, :]

    prop_h = _make_propagate_kernel(n, h1)
    prop_o = _make_propagate_kernel(n, out_d)
    bnd_h = _bounds_for(h1)
    bnd_o = _bounds_for(out_d)

    t1 = _dense_first(x, deg_flat, ws1, m1, b1r)
    a1 = prop_h(t1, row_p, col_p, bnd_h, zrows_h)
    t2 = _dense_mid(a1, t1, deg_flat, ws2, m2, b2r)
    a2 = prop_h(t2, row_p, col_p, bnd_h, zrows_h)
    t3 = _dense_mid(a2, t2, deg_flat, ws3, m3, b3r)
    a3 = prop_o(t3, row_p, col_p, bnd_o, zrows_o)
    return _dense_final(a3, t3, deg_flat)


# speedup vs baseline: 2.8904x; 1.1536x over previous
import functools

import jax
import jax.numpy as jnp
from jax import lax
from jax.experimental import pallas as pl
from jax.experimental.pallas import tpu as pltpu
from jax.experimental.pallas import tpu_sc as plsc

_EPS = 1e-5
_NC = 2
_NS = 16
_NW = _NC * _NS
_LANES = 16
_ND = 10240
_BD = 128
_RB = 1000


def _sc_mesh():
    return plsc.VectorSubcoreMesh(core_axis_name="c", subcore_axis_name="s")


def _make_degree_kernel(e, e_pad, nb_per_tile):
    nr = _ND // _LANES

    @functools.partial(
        pl.kernel,
        mesh=_sc_mesh(),
        out_type=jax.ShapeDtypeStruct((_NW, nr, _LANES), jnp.float32),
        scratch_types=[
            pltpu.VMEM((_BD,), jnp.int32),
            pltpu.VMEM((nr, _LANES), jnp.float32),
        ],
    )
    def degree_kernel(rows_hbm, zeros_hbm, out_hbm, idx_v, hist_v):
        c = lax.axis_index("c")
        s = lax.axis_index("s")
        w = s * _NC + c
        pltpu.sync_copy(zeros_hbm, hist_v)
        lanes = lax.iota(jnp.int32, _LANES)
        one = jnp.full((_LANES,), 1.0, jnp.float32)
        zero = jnp.zeros((_LANES,), jnp.float32)
        nb_all = (e + _BD - 1) // _BD
        nb_t = jnp.maximum((nb_all - w + _NW - 1) // _NW, 0)

        def _batch(m, carry):
            base = (w + m * _NW) * _BD
            pltpu.sync_copy(rows_hbm.at[pl.ds(base, _BD)], idx_v)
            for q in range(_BD // _LANES):
                pos = base + q * _LANES + lanes
                v = idx_v[pl.ds(q * _LANES, _LANES)]
                vm = jnp.where(pos < e, v, _ND - 1)
                for j in range(_LANES):
                    r = vm[j]
                    row = lax.shift_right_logical(r, 4)
                    lane = lax.bitwise_and(r, 15)
                    hist_v[row, :] += jnp.where(lanes == lane, one, zero)
            return carry

        lax.fori_loop(0, nb_t, _batch, 0)
        pltpu.sync_copy(hist_v, out_hbm.at[w])

    return degree_kernel


_PROP_CFG = {1024: (40, 32), 512: (80, 64), 272: (160, 112), 256: (160, 128)}


def _make_propagate_kernel(n, d):
    ct, bsz = _PROP_CFG[d]
    sweeps = _ND // (_NW * ct)
    assert _ND == sweeps * _NW * ct and ct % 8 == 0 and bsz % _LANES == 0

    @functools.partial(
        pl.kernel,
        mesh=_sc_mesh(),
        out_type=jax.ShapeDtypeStruct((_ND, d), jnp.float32),
        scratch_types=[
            pltpu.VMEM((bsz,), jnp.int32),
            pltpu.VMEM((bsz,), jnp.int32),
            pltpu.VMEM((bsz,), jnp.int32),
            pltpu.VMEM((bsz,), jnp.int32),
            pltpu.VMEM((bsz, d), jnp.float32),
            pltpu.VMEM((bsz, d), jnp.float32),
            pltpu.VMEM((1, _LANES), jnp.int32),
            pltpu.VMEM((ct + 1, d), jnp.float32),
            pltpu.SemaphoreType.DMA,
            pltpu.SemaphoreType.DMA,
        ],
    )
    def propagate_kernel(tsc_hbm, row_hbm, col_hbm, bnd_hbm, zrows_hbm,
                         out_hbm, ridx_a, cidx_a, ridx_b, cidx_b, msg_a,
                         msg_b, bvec_v, acc_v, sem_a, sem_b):
        c = lax.axis_index("c")
        s = lax.axis_index("s")
        w = s * _NC + c

        def _sweep(p, carry0):
            chunk = p * _NW + w
            base_rows = chunk * ct
            pltpu.sync_copy(zrows_hbm, acc_v.at[pl.ds(0, ct)])
            pltpu.sync_copy(bnd_hbm.at[chunk], bvec_v)
            bv = bvec_v[0, pl.ds(0, _LANES)]
            s_k = bv[0]
            e_k = bv[1]
            s_k8 = (s_k // 8) * 8
            nb = (e_k - s_k8 + bsz - 1) // bsz

            def _load_mask_start(m, ridx_v, cidx_v, msg_v, sem):
                base = s_k8 + m * bsz
                pltpu.sync_copy(row_hbm.at[pl.ds(base, bsz)], ridx_v)
                pltpu.sync_copy(col_hbm.at[pl.ds(base, bsz)], cidx_v)
                for q in range(bsz // _LANES):
                    pos = base + q * _LANES + lax.iota(jnp.int32, _LANES)
                    ok = (pos >= s_k) & (pos < e_k)
                    rv = ridx_v[pl.ds(q * _LANES, _LANES)]
                    cv = cidx_v[pl.ds(q * _LANES, _LANES)]
                    ridx_v[pl.ds(q * _LANES, _LANES)] = jnp.where(ok, rv, 0)
                    cidx_v[pl.ds(q * _LANES, _LANES)] = jnp.where(
                        ok, cv - base_rows, ct)
                pltpu.async_copy(tsc_hbm.at[ridx_v], msg_v, sem)

            def _rmw(cidx_v, msg_v):
                for g in range(bsz // _LANES):
                    cvec = cidx_v[pl.ds(g * _LANES, _LANES)]
                    rs = [cvec[j] for j in range(_LANES)]

                    def _qbody(q, carry2, g=g, rs=rs):
                        for j in range(_LANES):
                            plsc.addupdate(
                                acc_v.at[rs[j], pl.ds(q * _LANES, _LANES)],
                                msg_v[g * _LANES + j,
                                      pl.ds(q * _LANES, _LANES)])
                        return carry2

                    lax.fori_loop(0, d // _LANES, _qbody, 0)

            @pl.when(nb > 0)
            def _prologue():
                _load_mask_start(0, ridx_a, cidx_a, msg_a, sem_a)

            def _pair(it, carry):
                b0 = it * 2
                b1 = b0 + 1
                pltpu.make_async_copy(tsc_hbm.at[ridx_a], msg_a,
                                      sem_a).wait()

                @pl.when(b1 < nb)
                def _startb():
                    _load_mask_start(b1, ridx_b, cidx_b, msg_b, sem_b)

                _rmw(cidx_a, msg_a)

                @pl.when(b1 < nb)
                def _dob():
                    pltpu.make_async_copy(tsc_hbm.at[ridx_b], msg_b,
                                          sem_b).wait()

                    @pl.when(b0 + 2 < nb)
                    def _starta():
                        _load_mask_start(b0 + 2, ridx_a, cidx_a, msg_a,
                                         sem_a)

                    _rmw(cidx_b, msg_b)

                return carry

            lax.fori_loop(0, (nb + 1) // 2, _pair, 0)
            pltpu.sync_copy(acc_v.at[pl.ds(0, ct)],
                            out_hbm.at[pl.ds(base_rows, ct)])
            return carry0

        lax.fori_loop(0, sweeps, _sweep, 0)

    return propagate_kernel


def _bin_weights_body(w1_ref, w2_ref, w3_ref, s1_ref, s2_ref, s3_ref, m1_ref,
                      m2_ref, m3_ref):
    for wr, sr, mr in ((w1_ref, s1_ref, m1_ref), (w2_ref, s2_ref, m2_ref),
                       (w3_ref, s3_ref, m3_ref)):
        w = wr[...]
        mr[...] = jnp.mean(jnp.abs(w), axis=1)[None, :]
        sr[...] = jnp.sign(w).astype(jnp.bfloat16)


def _binarize_weights(W1, W2, W3):
    h, din = W1.shape
    out_d = W3.shape[0]
    return pl.pallas_call(
        _bin_weights_body,
        out_shape=[
            jax.ShapeDtypeStruct((h, din), jnp.bfloat16),
            jax.ShapeDtypeStruct((h, h), jnp.bfloat16),
            jax.ShapeDtypeStruct((out_d, h), jnp.bfloat16),
            jax.ShapeDtypeStruct((1, h), jnp.float32),
            jax.ShapeDtypeStruct((1, h), jnp.float32),
            jax.ShapeDtypeStruct((1, out_d), jnp.float32),
        ],
    )(W1, W2, W3)


def _dinv_of(deg_block):
    deg = jnp.sum(deg_block, axis=1) + 1.0
    return lax.rsqrt(deg)[:, None]


def _binlinear(h, ws_ref, wm_ref, b_ref, dinv):
    alpha = jnp.mean(jnp.abs(h), axis=1, keepdims=True)
    hs = jnp.sign(h).astype(jnp.bfloat16)
    acc = lax.dot_general(hs, ws_ref[...], (((1,), (1,)), ((), ())),
                          preferred_element_type=jnp.float32)
    return dinv * (acc * (alpha * wm_ref[...]) + b_ref[...])


def _d_first_body(x_ref, deg_ref, ws_ref, wm_ref, b_ref, out_ref):
    inv0 = (1.0 + _EPS) ** -0.5
    h = x_ref[...] * inv0
    out_ref[...] = _binlinear(h, ws_ref, wm_ref, b_ref, _dinv_of(deg_ref[...]))


def _d_mid_body(agg_ref, tsc_ref, deg_ref, ws_ref, wm_ref, b_ref, out_ref):
    dinv = _dinv_of(deg_ref[...])
    h = dinv * (agg_ref[...] + tsc_ref[...])
    out_ref[...] = _binlinear(h, ws_ref, wm_ref, b_ref, dinv)


def _d_final_body(agg_ref, tsc_ref, deg_ref, out_ref):
    dinv = _dinv_of(deg_ref[...])
    h = dinv * (agg_ref[...] + tsc_ref[...])
    mx = jnp.max(h, axis=1, keepdims=True)
    lse = jnp.log(jnp.sum(jnp.exp(h - mx), axis=1, keepdims=True)) + mx
    out_ref[...] = h - lse


def _row_spec(din):
    return pl.BlockSpec((_RB, din), lambda i: (i, 0))


def _deg_spec(n1):
    return pl.BlockSpec((_RB, _NW), lambda i: (i, 0))


def _full_spec(shape):
    return pl.BlockSpec(shape, lambda i: tuple(0 for _ in shape))


def _dense_first(x, deg16, ws, wm, b2d):
    n, din = x.shape
    dout = ws.shape[0]
    return pl.pallas_call(
        _d_first_body,
        grid=(n // _RB,),
        in_specs=[
            _row_spec(din),
            _deg_spec(n + 1),
            _full_spec(ws.shape),
            _full_spec(wm.shape),
            _full_spec(b2d.shape),
        ],
        out_specs=_row_spec(dout),
        out_shape=jax.ShapeDtypeStruct((n, dout), jnp.float32),
    )(x, deg16, ws, wm, b2d)


def _dense_mid(agg, tsc, deg16, ws, wm, b2d):
    n, din = tsc.shape
    dout = ws.shape[0]
    return pl.pallas_call(
        _d_mid_body,
        grid=(n // _RB,),
        in_specs=[
            _row_spec(din),
            _row_spec(din),
            _deg_spec(n + 1),
            _full_spec(ws.shape),
            _full_spec(wm.shape),
            _full_spec(b2d.shape),
        ],
        out_specs=_row_spec(dout),
        out_shape=jax.ShapeDtypeStruct((n, dout), jnp.float32),
    )(agg, tsc, deg16, ws, wm, b2d)


def _dense_final(agg, tsc, deg16):
    n, d = tsc.shape
    return pl.pallas_call(
        _d_final_body,
        grid=(n // _RB,),
        in_specs=[_row_spec(d), _row_spec(d), _deg_spec(n + 1)],
        out_specs=_row_spec(d),
        out_shape=jax.ShapeDtypeStruct((n, d), jnp.float32),
    )(agg, tsc, deg16)


def kernel(x, W1, b1, W2, b2, W3, b3, edge_index):
    n, _ = x.shape
    h1 = W1.shape[0]
    out_d = W3.shape[0]
    e = edge_index.shape[1]

    row = edge_index[0]
    col = edge_index[1]
    order = jnp.argsort(col)
    col_s = jnp.take(col, order)
    row_s = jnp.take(row, order)
    nb_deg = -(-e // _BD)
    e_pad = (-(-nb_deg // _NW)) * _NW * _BD
    pad = jnp.zeros((e_pad - e,), jnp.int32)
    row_p = jnp.concatenate([row_s, pad])
    col_p = jnp.concatenate([col_s, pad])

    def _bounds_for(d):
        ct = _PROP_CFG[d][0]
        nchunks = _ND // ct
        starts = jnp.searchsorted(
            col_s, jnp.arange(0, _ND + ct, ct, dtype=jnp.int32)).astype(
                jnp.int32)
        bnd = jnp.stack([starts[:-1], starts[1:]], axis=1)
        bnd = jnp.concatenate(
            [bnd, jnp.zeros((nchunks, 14), jnp.int32)], axis=1)
        return bnd.reshape(nchunks, 1, 16)

    zeros_deg = jnp.zeros((_ND // _LANES, _LANES), jnp.float32)
    zrows_h = jnp.zeros((_PROP_CFG[h1][0], h1), jnp.float32)
    zrows_o = jnp.zeros((_PROP_CFG[out_d][0], out_d), jnp.float32)

    deg_parts = _make_degree_kernel(e, e_pad, nb_deg)(row_p, zeros_deg)
    deg_flat = deg_parts.reshape(_NW, _ND).T
    ws1, ws2, ws3, m1, m2, m3 = _binarize_weights(W1, W2, W3)
    b1r, b2r, b3r = b1[None, :], b2[None, :], b3[None, :]

    prop_h = _make_propagate_kernel(n, h1)
    prop_o = _make_propagate_kernel(n, out_d)
    bnd_h = _bounds_for(h1)
    bnd_o = _bounds_for(out_d)

    t1 = _dense_first(x, deg_flat, ws1, m1, b1r)
    a1 = prop_h(t1, row_p, col_p, bnd_h, zrows_h)
    t2 = _dense_mid(a1, t1, deg_flat, ws2, m2, b2r)
    a2 = prop_h(t2, row_p, col_p, bnd_h, zrows_h)
    t3 = _dense_mid(a2, t2, deg_flat, ws3, m3, b3r)
    a3 = prop_o(t3, row_p, col_p, bnd_o, zrows_o)
    return _dense_final(a3, t3, deg_flat)


# speedup vs baseline: 3.5498x; 1.2281x over previous
import functools

import jax
import jax.numpy as jnp
from jax import lax
from jax.experimental import pallas as pl
from jax.experimental.pallas import tpu as pltpu
from jax.experimental.pallas import tpu_sc as plsc

_EPS = 1e-5
_NC = 2
_NS = 16
_NW = _NC * _NS
_LANES = 16
_ND = 10240
_BD = 128
_RB = 1000


def _sc_mesh():
    return plsc.VectorSubcoreMesh(core_axis_name="c", subcore_axis_name="s")


def _make_degree_kernel(e, e_pad, nb_per_tile):
    nr = _ND // _LANES

    @functools.partial(
        pl.kernel,
        mesh=_sc_mesh(),
        out_type=jax.ShapeDtypeStruct((_NW, nr, _LANES), jnp.float32),
        scratch_types=[
            pltpu.VMEM((_BD,), jnp.int32),
            pltpu.VMEM((nr, _LANES), jnp.float32),
        ],
    )
    def degree_kernel(rows_hbm, zeros_hbm, out_hbm, idx_v, hist_v):
        c = lax.axis_index("c")
        s = lax.axis_index("s")
        w = s * _NC + c
        pltpu.sync_copy(zeros_hbm, hist_v)
        lanes = lax.iota(jnp.int32, _LANES)
        one = jnp.full((_LANES,), 1.0, jnp.float32)
        zero = jnp.zeros((_LANES,), jnp.float32)
        nb_all = (e + _BD - 1) // _BD
        nb_t = jnp.maximum((nb_all - w + _NW - 1) // _NW, 0)

        def _batch(m, carry):
            base = (w + m * _NW) * _BD
            pltpu.sync_copy(rows_hbm.at[pl.ds(base, _BD)], idx_v)
            for q in range(_BD // _LANES):
                pos = base + q * _LANES + lanes
                v = idx_v[pl.ds(q * _LANES, _LANES)]
                vm = jnp.where(pos < e, v, _ND - 1)
                for j in range(_LANES):
                    r = vm[j]
                    row = lax.shift_right_logical(r, 4)
                    lane = lax.bitwise_and(r, 15)
                    hist_v[row, :] += jnp.where(lanes == lane, one, zero)
            return carry

        lax.fori_loop(0, nb_t, _batch, 0)
        pltpu.sync_copy(hist_v, out_hbm.at[w])

    return degree_kernel


_PROP_CFG = {1024: (40, 32), 512: (80, 64), 384: (80, 96), 256: (160, 128)}
_DE = 384


def _make_propagate_kernel(n, d):
    ct, bsz = _PROP_CFG[d]
    sweeps = _ND // (_NW * ct)
    assert _ND == sweeps * _NW * ct and ct % 8 == 0 and bsz % _LANES == 0

    @functools.partial(
        pl.kernel,
        mesh=_sc_mesh(),
        out_type=jax.ShapeDtypeStruct((_ND, d), jnp.float32),
        scratch_types=[
            pltpu.VMEM((bsz,), jnp.int32),
            pltpu.VMEM((bsz,), jnp.int32),
            pltpu.VMEM((bsz,), jnp.int32),
            pltpu.VMEM((bsz,), jnp.int32),
            pltpu.VMEM((bsz, d), jnp.float32),
            pltpu.VMEM((bsz, d), jnp.float32),
            pltpu.VMEM((1, _LANES), jnp.int32),
            pltpu.VMEM((ct + 1, d), jnp.float32),
            pltpu.SemaphoreType.DMA,
            pltpu.SemaphoreType.DMA,
        ],
    )
    def propagate_kernel(tsc_hbm, row_hbm, col_hbm, bnd_hbm, zrows_hbm,
                         out_hbm, ridx_a, cidx_a, ridx_b, cidx_b, msg_a,
                         msg_b, bvec_v, acc_v, sem_a, sem_b):
        c = lax.axis_index("c")
        s = lax.axis_index("s")
        w = s * _NC + c

        def _sweep(p, carry0):
            chunk = p * _NW + w
            base_rows = chunk * ct
            pltpu.sync_copy(zrows_hbm, acc_v.at[pl.ds(0, ct)])
            pltpu.sync_copy(bnd_hbm.at[chunk], bvec_v)
            bv = bvec_v[0, pl.ds(0, _LANES)]
            s_k = bv[0]
            e_k = bv[1]
            s_k8 = (s_k // 8) * 8
            nb = (e_k - s_k8 + bsz - 1) // bsz

            def _load_mask_start(m, ridx_v, cidx_v, msg_v, sem):
                base = s_k8 + m * bsz
                pltpu.sync_copy(row_hbm.at[pl.ds(base, bsz)], ridx_v)
                pltpu.sync_copy(col_hbm.at[pl.ds(base, bsz)], cidx_v)
                for q in range(bsz // _LANES):
                    pos = base + q * _LANES + lax.iota(jnp.int32, _LANES)
                    ok = (pos >= s_k) & (pos < e_k)
                    rv = ridx_v[pl.ds(q * _LANES, _LANES)]
                    cv = cidx_v[pl.ds(q * _LANES, _LANES)]
                    ridx_v[pl.ds(q * _LANES, _LANES)] = jnp.where(ok, rv, 0)
                    cidx_v[pl.ds(q * _LANES, _LANES)] = jnp.where(
                        ok, cv - base_rows, ct)
                pltpu.async_copy(tsc_hbm.at[ridx_v], msg_v, sem)

            def _rmw(cidx_v, msg_v):
                for g in range(bsz // _LANES):
                    cvec = cidx_v[pl.ds(g * _LANES, _LANES)]
                    rs = [cvec[j] for j in range(_LANES)]

                    def _qbody(q, carry2, g=g, rs=rs):
                        for j in range(_LANES):
                            plsc.addupdate(
                                acc_v.at[rs[j], pl.ds(q * _LANES, _LANES)],
                                msg_v[g * _LANES + j,
                                      pl.ds(q * _LANES, _LANES)])
                        return carry2

                    lax.fori_loop(0, d // _LANES, _qbody, 0)

            @pl.when(nb > 0)
            def _prologue():
                _load_mask_start(0, ridx_a, cidx_a, msg_a, sem_a)

            def _pair(it, carry):
                b0 = it * 2
                b1 = b0 + 1
                pltpu.make_async_copy(tsc_hbm.at[ridx_a], msg_a,
                                      sem_a).wait()

                @pl.when(b1 < nb)
                def _startb():
                    _load_mask_start(b1, ridx_b, cidx_b, msg_b, sem_b)

                _rmw(cidx_a, msg_a)

                @pl.when(b1 < nb)
                def _dob():
                    pltpu.make_async_copy(tsc_hbm.at[ridx_b], msg_b,
                                          sem_b).wait()

                    @pl.when(b0 + 2 < nb)
                    def _starta():
                        _load_mask_start(b0 + 2, ridx_a, cidx_a, msg_a,
                                         sem_a)

                    _rmw(cidx_b, msg_b)

                return carry

            lax.fori_loop(0, (nb + 1) // 2, _pair, 0)
            pltpu.sync_copy(acc_v.at[pl.ds(0, ct)],
                            out_hbm.at[pl.ds(base_rows, ct)])
            return carry0

        lax.fori_loop(0, sweeps, _sweep, 0)

    return propagate_kernel


def _bin_weights_body(w1_ref, w2_ref, w3_ref, s1_ref, s2_ref, s3_ref, m1_ref,
                      m2_ref, m3_ref):
    for wr, sr, mr in ((w1_ref, s1_ref, m1_ref), (w2_ref, s2_ref, m2_ref),
                       (w3_ref, s3_ref, m3_ref)):
        w = wr[...]
        mr[...] = jnp.mean(jnp.abs(w), axis=1)[None, :]
        sr[...] = jnp.sign(w).astype(jnp.bfloat16)


def _binarize_weights(W1, W2, W3):
    h, din = W1.shape
    out_d = W3.shape[0]
    return pl.pallas_call(
        _bin_weights_body,
        out_shape=[
            jax.ShapeDtypeStruct((h, din), jnp.bfloat16),
            jax.ShapeDtypeStruct((h, h), jnp.bfloat16),
            jax.ShapeDtypeStruct((out_d, h), jnp.bfloat16),
            jax.ShapeDtypeStruct((1, h), jnp.float32),
            jax.ShapeDtypeStruct((1, h), jnp.float32),
            jax.ShapeDtypeStruct((1, out_d), jnp.float32),
        ],
    )(W1, W2, W3)


def _dinv_of(deg_block):
    deg = jnp.sum(deg_block, axis=1) + 1.0
    return lax.rsqrt(deg)[:, None]


def _binlinear(h, ws_ref, wm_ref, b_ref, dinv):
    alpha = jnp.mean(jnp.abs(h), axis=1, keepdims=True)
    hs = jnp.sign(h).astype(jnp.bfloat16)
    acc = lax.dot_general(hs, ws_ref[...], (((1,), (1,)), ((), ())),
                          preferred_element_type=jnp.float32)
    return dinv * (acc * (alpha * wm_ref[...]) + b_ref[...])


def _d_pre_body(x_ref, deg_ref, out_ref):
    inv0 = (1.0 + _EPS) ** -0.5
    h = x_ref[...] * inv0
    dinv = _dinv_of(deg_ref[...])
    alpha = jnp.mean(jnp.abs(h), axis=1, keepdims=True)
    val = dinv * (alpha * jnp.sign(h))
    pad = jnp.zeros((val.shape[0], _DE - val.shape[1] - 1), jnp.float32)
    out_ref[...] = jnp.concatenate([val, dinv, pad], axis=1)


def _d_first2_body(agg_ref, pre_ref, deg_ref, ws1_ref, wm1_ref, b1_ref,
                   ws2_ref, wm2_ref, b2_ref, out_ref):
    din = ws1_ref.shape[1]
    dinv = _dinv_of(deg_ref[...])
    pe = dinv * (agg_ref[...] + pre_ref[...])
    p1 = pe[:, :din]
    sv = pe[:, din][:, None]
    t1 = lax.dot_general(p1, ws1_ref[...].astype(jnp.float32),
                         (((1,), (1,)), ((), ())),
                         preferred_element_type=jnp.float32)
    h1 = t1 * wm1_ref[...] + sv * b1_ref[...]
    out_ref[...] = _binlinear(h1, ws2_ref, wm2_ref, b2_ref, dinv)


def _d_mid_body(agg_ref, tsc_ref, deg_ref, ws_ref, wm_ref, b_ref, out_ref):
    dinv = _dinv_of(deg_ref[...])
    h = dinv * (agg_ref[...] + tsc_ref[...])
    out_ref[...] = _binlinear(h, ws_ref, wm_ref, b_ref, dinv)


def _d_final_body(agg_ref, tsc_ref, deg_ref, out_ref):
    dinv = _dinv_of(deg_ref[...])
    h = dinv * (agg_ref[...] + tsc_ref[...])
    mx = jnp.max(h, axis=1, keepdims=True)
    lse = jnp.log(jnp.sum(jnp.exp(h - mx), axis=1, keepdims=True)) + mx
    out_ref[...] = h - lse


def _row_spec(din):
    return pl.BlockSpec((_RB, din), lambda i: (i, 0))


def _deg_spec(n1):
    return pl.BlockSpec((_RB, _NW), lambda i: (i, 0))


def _full_spec(shape):
    return pl.BlockSpec(shape, lambda i: tuple(0 for _ in shape))


def _dense_pre(x, deg16):
    n, din = x.shape
    return pl.pallas_call(
        _d_pre_body,
        grid=(n // _RB,),
        in_specs=[_row_spec(din), _deg_spec(n + 1)],
        out_specs=_row_spec(_DE),
        out_shape=jax.ShapeDtypeStruct((n, _DE), jnp.float32),
    )(x, deg16)


def _dense_first2(agg, pre, deg16, ws1, wm1, b1r, ws2, wm2, b2r):
    n = pre.shape[0]
    dout = ws2.shape[0]
    return pl.pallas_call(
        _d_first2_body,
        grid=(n // _RB,),
        in_specs=[
            _row_spec(_DE),
            _row_spec(_DE),
            _deg_spec(n + 1),
            _full_spec(ws1.shape),
            _full_spec(wm1.shape),
            _full_spec(b1r.shape),
            _full_spec(ws2.shape),
            _full_spec(wm2.shape),
            _full_spec(b2r.shape),
        ],
        out_specs=_row_spec(dout),
        out_shape=jax.ShapeDtypeStruct((n, dout), jnp.float32),
    )(agg, pre, deg16, ws1, wm1, b1r, ws2, wm2, b2r)


def _dense_mid(agg, tsc, deg16, ws, wm, b2d):
    n, din = tsc.shape
    dout = ws.shape[0]
    return pl.pallas_call(
        _d_mid_body,
        grid=(n // _RB,),
        in_specs=[
            _row_spec(din),
            _row_spec(din),
            _deg_spec(n + 1),
            _full_spec(ws.shape),
            _full_spec(wm.shape),
            _full_spec(b2d.shape),
        ],
        out_specs=_row_spec(dout),
        out_shape=jax.ShapeDtypeStruct((n, dout), jnp.float32),
    )(agg, tsc, deg16, ws, wm, b2d)


def _dense_final(agg, tsc, deg16):
    n, d = tsc.shape
    return pl.pallas_call(
        _d_final_body,
        grid=(n // _RB,),
        in_specs=[_row_spec(d), _row_spec(d), _deg_spec(n + 1)],
        out_specs=_row_spec(d),
        out_shape=jax.ShapeDtypeStruct((n, d), jnp.float32),
    )(agg, tsc, deg16)


def kernel(x, W1, b1, W2, b2, W3, b3, edge_index):
    n, _ = x.shape
    h1 = W1.shape[0]
    out_d = W3.shape[0]
    e = edge_index.shape[1]

    row = edge_index[0]
    col = edge_index[1]
    order = jnp.argsort(col)
    col_s = jnp.take(col, order)
    row_s = jnp.take(row, order)
    nb_deg = -(-e // _BD)
    e_pad = (-(-nb_deg // _NW)) * _NW * _BD
    pad = jnp.zeros((e_pad - e,), jnp.int32)
    row_p = jnp.concatenate([row_s, pad])
    col_p = jnp.concatenate([col_s, pad])

    def _bounds_for(d):
        ct = _PROP_CFG[d][0]
        nchunks = _ND // ct
        starts = jnp.searchsorted(
            col_s, jnp.arange(0, _ND + ct, ct, dtype=jnp.int32)).astype(
                jnp.int32)
        bnd = jnp.stack([starts[:-1], starts[1:]], axis=1)
        bnd = jnp.concatenate(
            [bnd, jnp.zeros((nchunks, 14), jnp.int32)], axis=1)
        return bnd.reshape(nchunks, 1, 16)

    zeros_deg = jnp.zeros((_ND // _LANES, _LANES), jnp.float32)
    zrows_h = jnp.zeros((_PROP_CFG[h1][0], h1), jnp.float32)
    zrows_o = jnp.zeros((_PROP_CFG[out_d][0], out_d), jnp.float32)

    deg_parts = _make_degree_kernel(e, e_pad, nb_deg)(row_p, zeros_deg)
    deg_flat = deg_parts.reshape(_NW, _ND).T
    ws1, ws2, ws3, m1, m2, m3 = _binarize_weights(W1, W2, W3)
    b1r, b2r, b3r = b1[None, :], b2[None, :], b3[None, :]

    prop_e = _make_propagate_kernel(n, _DE)
    prop_h = _make_propagate_kernel(n, h1)
    prop_o = _make_propagate_kernel(n, out_d)
    bnd_e = _bounds_for(_DE)
    bnd_h = _bounds_for(h1)
    bnd_o = _bounds_for(out_d)
    zrows_e = jnp.zeros((_PROP_CFG[_DE][0], _DE), jnp.float32)

    pre0 = _dense_pre(x, deg_flat)
    a0 = prop_e(pre0, row_p, col_p, bnd_e, zrows_e)
    t2 = _dense_first2(a0, pre0, deg_flat, ws1, m1, b1r, ws2, m2, b2r)
    a2 = prop_h(t2, row_p, col_p, bnd_h, zrows_h)
    t3 = _dense_mid(a2, t2, deg_flat, ws3, m3, b3r)
    a3 = prop_o(t3, row_p, col_p, bnd_o, zrows_o)
    return _dense_final(a3, t3, deg_flat)


# speedup vs baseline: 5.1542x; 1.4520x over previous
import functools

import jax
import jax.numpy as jnp
from jax import lax
from jax.experimental import pallas as pl
from jax.experimental.pallas import tpu as pltpu
from jax.experimental.pallas import tpu_sc as plsc

_EPS = 1e-5
_NC = 2
_NS = 16
_NW = _NC * _NS
_LANES = 16
_ND = 10240
_BD = 128
_RB = 1000


def _sc_mesh():
    return plsc.VectorSubcoreMesh(core_axis_name="c", subcore_axis_name="s")


def _make_degree_kernel(e, e_pad, nb_per_tile):
    nr = _ND // _LANES

    @functools.partial(
        pl.kernel,
        mesh=_sc_mesh(),
        out_type=jax.ShapeDtypeStruct((_NW, nr, _LANES), jnp.float32),
        scratch_types=[
            pltpu.VMEM((_BD,), jnp.int32),
            pltpu.VMEM((nr, _LANES), jnp.float32),
        ],
    )
    def degree_kernel(rows_hbm, zeros_hbm, out_hbm, idx_v, hist_v):
        c = lax.axis_index("c")
        s = lax.axis_index("s")
        w = s * _NC + c
        pltpu.sync_copy(zeros_hbm, hist_v)
        lanes = lax.iota(jnp.int32, _LANES)
        one = jnp.full((_LANES,), 1.0, jnp.float32)
        zero = jnp.zeros((_LANES,), jnp.float32)
        nb_all = (e + _BD - 1) // _BD
        nb_t = jnp.maximum((nb_all - w + _NW - 1) // _NW, 0)

        def _batch(m, carry):
            base = (w + m * _NW) * _BD
            pltpu.sync_copy(rows_hbm.at[pl.ds(base, _BD)], idx_v)
            for q in range(_BD // _LANES):
                pos = base + q * _LANES + lanes
                v = idx_v[pl.ds(q * _LANES, _LANES)]
                vm = jnp.where(pos < e, v, _ND - 1)
                for j in range(_LANES):
                    r = vm[j]
                    row = lax.shift_right_logical(r, 4)
                    lane = lax.bitwise_and(r, 15)
                    hist_v[row, :] += jnp.where(lanes == lane, one, zero)
            return carry

        lax.fori_loop(0, nb_t, _batch, 0)
        pltpu.sync_copy(hist_v, out_hbm.at[w])

    return degree_kernel


_PROP_CFG = {1024: (40, 32), 512: (80, 64), 384: (80, 96), 256: (160, 128)}
_DE = 384


def _make_propagate_kernel(n, d):
    ct, bsz = _PROP_CFG[d]
    sweeps = _ND // (_NW * ct)
    assert _ND == sweeps * _NW * ct and ct % 8 == 0 and bsz % _LANES == 0

    @functools.partial(
        pl.kernel,
        mesh=_sc_mesh(),
        out_type=jax.ShapeDtypeStruct((_ND, d), jnp.float32),
        scratch_types=[
            pltpu.VMEM((bsz,), jnp.int32),
            pltpu.VMEM((bsz,), jnp.int32),
            pltpu.VMEM((bsz,), jnp.int32),
            pltpu.VMEM((bsz,), jnp.int32),
            pltpu.VMEM((bsz, d), jnp.float32),
            pltpu.VMEM((bsz, d), jnp.float32),
            pltpu.VMEM((1, _LANES), jnp.int32),
            pltpu.VMEM((ct + 1, d), jnp.float32),
            pltpu.SemaphoreType.DMA,
            pltpu.SemaphoreType.DMA,
        ],
    )
    def propagate_kernel(tsc_hbm, row_hbm, col_hbm, bnd_hbm, zrows_hbm,
                         out_hbm, ridx_a, cidx_a, ridx_b, cidx_b, msg_a,
                         msg_b, bvec_v, acc_v, sem_a, sem_b):
        c = lax.axis_index("c")
        s = lax.axis_index("s")
        w = s * _NC + c

        def _sweep(p, carry0):
            chunk = p * _NW + w
            base_rows = chunk * ct
            pltpu.sync_copy(zrows_hbm, acc_v.at[pl.ds(0, ct)])
            pltpu.sync_copy(bnd_hbm.at[chunk], bvec_v)
            bv = bvec_v[0, pl.ds(0, _LANES)]
            s_k = bv[0]
            e_k = bv[1]
            s_k8 = (s_k // 8) * 8
            nb = (e_k - s_k8 + bsz - 1) // bsz

            def _load_mask_start(m, ridx_v, cidx_v, msg_v, sem):
                base = s_k8 + m * bsz
                pltpu.sync_copy(row_hbm.at[pl.ds(base, bsz)], ridx_v)
                pltpu.sync_copy(col_hbm.at[pl.ds(base, bsz)], cidx_v)
                for q in range(bsz // _LANES):
                    pos = base + q * _LANES + lax.iota(jnp.int32, _LANES)
                    ok = (pos >= s_k) & (pos < e_k)
                    rv = ridx_v[pl.ds(q * _LANES, _LANES)]
                    cv = cidx_v[pl.ds(q * _LANES, _LANES)]
                    ridx_v[pl.ds(q * _LANES, _LANES)] = jnp.where(ok, rv, 0)
                    cidx_v[pl.ds(q * _LANES, _LANES)] = jnp.where(
                        ok, cv - base_rows, ct)
                pltpu.async_copy(tsc_hbm.at[ridx_v], msg_v, sem)

            def _rmw(cidx_v, msg_v):
                for g in range(bsz // _LANES):
                    cvec = cidx_v[pl.ds(g * _LANES, _LANES)]
                    rs = [cvec[j] for j in range(_LANES)]

                    @plsc.parallel_loop(0, d // _LANES, 1, unroll=4)
                    def _qbody(q, g=g, rs=rs):
                        for j in range(_LANES):
                            plsc.addupdate(
                                acc_v.at[rs[j], pl.ds(q * _LANES, _LANES)],
                                msg_v[g * _LANES + j,
                                      pl.ds(q * _LANES, _LANES)])

            @pl.when(nb > 0)
            def _prologue():
                _load_mask_start(0, ridx_a, cidx_a, msg_a, sem_a)

            def _pair(it, carry):
                b0 = it * 2
                b1 = b0 + 1
                pltpu.make_async_copy(tsc_hbm.at[ridx_a], msg_a,
                                      sem_a).wait()

                @pl.when(b1 < nb)
                def _startb():
                    _load_mask_start(b1, ridx_b, cidx_b, msg_b, sem_b)

                _rmw(cidx_a, msg_a)

                @pl.when(b1 < nb)
                def _dob():
                    pltpu.make_async_copy(tsc_hbm.at[ridx_b], msg_b,
                                          sem_b).wait()

                    @pl.when(b0 + 2 < nb)
                    def _starta():
                        _load_mask_start(b0 + 2, ridx_a, cidx_a, msg_a,
                                         sem_a)

                    _rmw(cidx_b, msg_b)

                return carry

            lax.fori_loop(0, (nb + 1) // 2, _pair, 0)
            pltpu.sync_copy(acc_v.at[pl.ds(0, ct)],
                            out_hbm.at[pl.ds(base_rows, ct)])
            return carry0

        lax.fori_loop(0, sweeps, _sweep, 0)

    return propagate_kernel


def _bin_weights_body(w1_ref, w2_ref, w3_ref, s1_ref, s2_ref, s3_ref, m1_ref,
                      m2_ref, m3_ref):
    for wr, sr, mr in ((w1_ref, s1_ref, m1_ref), (w2_ref, s2_ref, m2_ref),
                       (w3_ref, s3_ref, m3_ref)):
        w = wr[...]
        mr[...] = jnp.mean(jnp.abs(w), axis=1)[None, :]
        sr[...] = jnp.sign(w).astype(jnp.bfloat16)


def _binarize_weights(W1, W2, W3):
    h, din = W1.shape
    out_d = W3.shape[0]
    return pl.pallas_call(
        _bin_weights_body,
        out_shape=[
            jax.ShapeDtypeStruct((h, din), jnp.bfloat16),
            jax.ShapeDtypeStruct((h, h), jnp.bfloat16),
            jax.ShapeDtypeStruct((out_d, h), jnp.bfloat16),
            jax.ShapeDtypeStruct((1, h), jnp.float32),
            jax.ShapeDtypeStruct((1, h), jnp.float32),
            jax.ShapeDtypeStruct((1, out_d), jnp.float32),
        ],
    )(W1, W2, W3)


def _dinv_of(deg_block):
    deg = jnp.sum(deg_block, axis=1) + 1.0
    return lax.rsqrt(deg)[:, None]


def _binlinear(h, ws_ref, wm_ref, b_ref, dinv):
    alpha = jnp.mean(jnp.abs(h), axis=1, keepdims=True)
    hs = jnp.sign(h).astype(jnp.bfloat16)
    acc = lax.dot_general(hs, ws_ref[...], (((1,), (1,)), ((), ())),
                          preferred_element_type=jnp.float32)
    return dinv * (acc * (alpha * wm_ref[...]) + b_ref[...])


def _d_pre_body(x_ref, deg_ref, out_ref):
    inv0 = (1.0 + _EPS) ** -0.5
    h = x_ref[...] * inv0
    dinv = _dinv_of(deg_ref[...])
    alpha = jnp.mean(jnp.abs(h), axis=1, keepdims=True)
    val = dinv * (alpha * jnp.sign(h))
    pad = jnp.zeros((val.shape[0], _DE - val.shape[1] - 1), jnp.float32)
    out_ref[...] = jnp.concatenate([val, dinv, pad], axis=1)


def _d_first2_body(agg_ref, pre_ref, deg_ref, ws1_ref, wm1_ref, b1_ref,
                   ws2_ref, wm2_ref, b2_ref, out_ref):
    din = ws1_ref.shape[1]
    dinv = _dinv_of(deg_ref[...])
    pe = dinv * (agg_ref[...] + pre_ref[...])
    p1 = pe[:, :din]
    sv = pe[:, din][:, None]
    t1 = lax.dot_general(p1, ws1_ref[...].astype(jnp.float32),
                         (((1,), (1,)), ((), ())),
                         preferred_element_type=jnp.float32)
    h1 = t1 * wm1_ref[...] + sv * b1_ref[...]
    out_ref[...] = _binlinear(h1, ws2_ref, wm2_ref, b2_ref, dinv)


def _d_mid_body(agg_ref, tsc_ref, deg_ref, ws_ref, wm_ref, b_ref, out_ref):
    dinv = _dinv_of(deg_ref[...])
    h = dinv * (agg_ref[...] + tsc_ref[...])
    out_ref[...] = _binlinear(h, ws_ref, wm_ref, b_ref, dinv)


def _d_final_body(agg_ref, tsc_ref, deg_ref, out_ref):
    dinv = _dinv_of(deg_ref[...])
    h = dinv * (agg_ref[...] + tsc_ref[...])
    mx = jnp.max(h, axis=1, keepdims=True)
    lse = jnp.log(jnp.sum(jnp.exp(h - mx), axis=1, keepdims=True)) + mx
    out_ref[...] = h - lse


def _row_spec(din):
    return pl.BlockSpec((_RB, din), lambda i: (i, 0))


def _deg_spec(n1):
    return pl.BlockSpec((_RB, _NW), lambda i: (i, 0))


def _full_spec(shape):
    return pl.BlockSpec(shape, lambda i: tuple(0 for _ in shape))


def _dense_pre(x, deg16):
    n, din = x.shape
    return pl.pallas_call(
        _d_pre_body,
        grid=(n // _RB,),
        in_specs=[_row_spec(din), _deg_spec(n + 1)],
        out_specs=_row_spec(_DE),
        out_shape=jax.ShapeDtypeStruct((n, _DE), jnp.float32),
    )(x, deg16)


def _dense_first2(agg, pre, deg16, ws1, wm1, b1r, ws2, wm2, b2r):
    n = pre.shape[0]
    dout = ws2.shape[0]
    return pl.pallas_call(
        _d_first2_body,
        grid=(n // _RB,),
        in_specs=[
            _row_spec(_DE),
            _row_spec(_DE),
            _deg_spec(n + 1),
            _full_spec(ws1.shape),
            _full_spec(wm1.shape),
            _full_spec(b1r.shape),
            _full_spec(ws2.shape),
            _full_spec(wm2.shape),
            _full_spec(b2r.shape),
        ],
        out_specs=_row_spec(dout),
        out_shape=jax.ShapeDtypeStruct((n, dout), jnp.float32),
    )(agg, pre, deg16, ws1, wm1, b1r, ws2, wm2, b2r)


def _dense_mid(agg, tsc, deg16, ws, wm, b2d):
    n, din = tsc.shape
    dout = ws.shape[0]
    return pl.pallas_call(
        _d_mid_body,
        grid=(n // _RB,),
        in_specs=[
            _row_spec(din),
            _row_spec(din),
            _deg_spec(n + 1),
            _full_spec(ws.shape),
            _full_spec(wm.shape),
            _full_spec(b2d.shape),
        ],
        out_specs=_row_spec(dout),
        out_shape=jax.ShapeDtypeStruct((n, dout), jnp.float32),
    )(agg, tsc, deg16, ws, wm, b2d)


def _dense_final(agg, tsc, deg16):
    n, d = tsc.shape
    return pl.pallas_call(
        _d_final_body,
        grid=(n // _RB,),
        in_specs=[_row_spec(d), _row_spec(d), _deg_spec(n + 1)],
        out_specs=_row_spec(d),
        out_shape=jax.ShapeDtypeStruct((n, d), jnp.float32),
    )(agg, tsc, deg16)


def kernel(x, W1, b1, W2, b2, W3, b3, edge_index):
    n, _ = x.shape
    h1 = W1.shape[0]
    out_d = W3.shape[0]
    e = edge_index.shape[1]

    row = edge_index[0]
    col = edge_index[1]
    order = jnp.argsort(col)
    col_s = jnp.take(col, order)
    row_s = jnp.take(row, order)
    nb_deg = -(-e // _BD)
    e_pad = (-(-nb_deg // _NW)) * _NW * _BD
    pad = jnp.zeros((e_pad - e,), jnp.int32)
    row_p = jnp.concatenate([row_s, pad])
    col_p = jnp.concatenate([col_s, pad])

    def _bounds_for(d):
        ct = _PROP_CFG[d][0]
        nchunks = _ND // ct
        starts = jnp.searchsorted(
            col_s, jnp.arange(0, _ND + ct, ct, dtype=jnp.int32)).astype(
                jnp.int32)
        bnd = jnp.stack([starts[:-1], starts[1:]], axis=1)
        bnd = jnp.concatenate(
            [bnd, jnp.zeros((nchunks, 14), jnp.int32)], axis=1)
        return bnd.reshape(nchunks, 1, 16)

    zeros_deg = jnp.zeros((_ND // _LANES, _LANES), jnp.float32)
    zrows_h = jnp.zeros((_PROP_CFG[h1][0], h1), jnp.float32)
    zrows_o = jnp.zeros((_PROP_CFG[out_d][0], out_d), jnp.float32)

    deg_parts = _make_degree_kernel(e, e_pad, nb_deg)(row_p, zeros_deg)
    deg_flat = deg_parts.reshape(_NW, _ND).T
    ws1, ws2, ws3, m1, m2, m3 = _binarize_weights(W1, W2, W3)
    b1r, b2r, b3r = b1[None, :], b2[None, :], b3[None, :]

    prop_e = _make_propagate_kernel(n, _DE)
    prop_h = _make_propagate_kernel(n, h1)
    prop_o = _make_propagate_kernel(n, out_d)
    bnd_e = _bounds_for(_DE)
    bnd_h = _bounds_for(h1)
    bnd_o = _bounds_for(out_d)
    zrows_e = jnp.zeros((_PROP_CFG[_DE][0], _DE), jnp.float32)

    pre0 = _dense_pre(x, deg_flat)
    a0 = prop_e(pre0, row_p, col_p, bnd_e, zrows_e)
    t2 = _dense_first2(a0, pre0, deg_flat, ws1, m1, b1r, ws2, m2, b2r)
    a2 = prop_h(t2, row_p, col_p, bnd_h, zrows_h)
    t3 = _dense_mid(a2, t2, deg_flat, ws3, m3, b3r)
    a3 = prop_o(t3, row_p, col_p, bnd_o, zrows_o)
    return _dense_final(a3, t3, deg_flat)


# speedup vs baseline: 5.3238x; 1.0329x over previous
import functools

import jax
import jax.numpy as jnp
from jax import lax
from jax.experimental import pallas as pl
from jax.experimental.pallas import tpu as pltpu
from jax.experimental.pallas import tpu_sc as plsc

_EPS = 1e-5
_NC = 2
_NS = 16
_NW = _NC * _NS
_LANES = 16
_ND = 10240
_BD = 128
_RB = 1000


def _sc_mesh():
    return plsc.VectorSubcoreMesh(core_axis_name="c", subcore_axis_name="s")


def _make_degree_kernel(e, e_pad, nb_per_tile):
    nr = _ND // _LANES

    @functools.partial(
        pl.kernel,
        mesh=_sc_mesh(),
        out_type=jax.ShapeDtypeStruct((_NW, nr, _LANES), jnp.float32),
        scratch_types=[
            pltpu.VMEM((_BD,), jnp.int32),
            pltpu.VMEM((nr, _LANES), jnp.float32),
        ],
    )
    def degree_kernel(rows_hbm, zeros_hbm, out_hbm, idx_v, hist_v):
        c = lax.axis_index("c")
        s = lax.axis_index("s")
        w = s * _NC + c
        pltpu.sync_copy(zeros_hbm, hist_v)
        lanes = lax.iota(jnp.int32, _LANES)
        one = jnp.full((_LANES,), 1.0, jnp.float32)
        zero = jnp.zeros((_LANES,), jnp.float32)
        nb_all = (e + _BD - 1) // _BD
        nb_t = jnp.maximum((nb_all - w + _NW - 1) // _NW, 0)

        def _batch(m, carry):
            base = (w + m * _NW) * _BD
            pltpu.sync_copy(rows_hbm.at[pl.ds(base, _BD)], idx_v)
            for q in range(_BD // _LANES):
                pos = base + q * _LANES + lanes
                v = idx_v[pl.ds(q * _LANES, _LANES)]
                vm = jnp.where(pos < e, v, _ND - 1)
                for j in range(_LANES):
                    r = vm[j]
                    row = lax.shift_right_logical(r, 4)
                    lane = lax.bitwise_and(r, 15)
                    hist_v[row, :] += jnp.where(lanes == lane, one, zero)
            return carry

        lax.fori_loop(0, nb_t, _batch, 0)
        pltpu.sync_copy(hist_v, out_hbm.at[w])

    return degree_kernel


_PROP_CFG = {1024: (16, 48), 512: (40, 80), 384: (40, 112), 256: (160, 128)}
_DE = 384


def _make_propagate_kernel(n, d):
    ct, bsz = _PROP_CFG[d]
    sweeps = _ND // (_NW * ct)
    assert _ND == sweeps * _NW * ct and ct % 8 == 0 and bsz % _LANES == 0

    @functools.partial(
        pl.kernel,
        mesh=_sc_mesh(),
        out_type=jax.ShapeDtypeStruct((_ND, d), jnp.float32),
        scratch_types=[
            pltpu.VMEM((bsz,), jnp.int32),
            pltpu.VMEM((bsz,), jnp.int32),
            pltpu.VMEM((bsz,), jnp.int32),
            pltpu.VMEM((bsz,), jnp.int32),
            pltpu.VMEM((bsz, d), jnp.float32),
            pltpu.VMEM((bsz, d), jnp.float32),
            pltpu.VMEM((1, _LANES), jnp.int32),
            pltpu.VMEM((ct + 1, d), jnp.float32),
            pltpu.SemaphoreType.DMA,
            pltpu.SemaphoreType.DMA,
        ],
    )
    def propagate_kernel(tsc_hbm, row_hbm, col_hbm, bnd_hbm, zrows_hbm,
                         out_hbm, ridx_a, cidx_a, ridx_b, cidx_b, msg_a,
                         msg_b, bvec_v, acc_v, sem_a, sem_b):
        c = lax.axis_index("c")
        s = lax.axis_index("s")
        w = s * _NC + c

        def _sweep(p, carry0):
            chunk = p * _NW + w
            base_rows = chunk * ct
            pltpu.sync_copy(zrows_hbm, acc_v.at[pl.ds(0, ct)])
            pltpu.sync_copy(bnd_hbm.at[chunk], bvec_v)
            bv = bvec_v[0, pl.ds(0, _LANES)]
            s_k = bv[0]
            e_k = bv[1]
            s_k8 = (s_k // 8) * 8
            nb = (e_k - s_k8 + bsz - 1) // bsz

            def _load_mask_start(m, ridx_v, cidx_v, msg_v, sem):
                base = s_k8 + m * bsz
                pltpu.sync_copy(row_hbm.at[pl.ds(base, bsz)], ridx_v)
                pltpu.sync_copy(col_hbm.at[pl.ds(base, bsz)], cidx_v)
                for q in range(bsz // _LANES):
                    pos = base + q * _LANES + lax.iota(jnp.int32, _LANES)
                    ok = (pos >= s_k) & (pos < e_k)
                    cv = cidx_v[pl.ds(q * _LANES, _LANES)]
                    cidx_v[pl.ds(q * _LANES, _LANES)] = jnp.where(
                        ok, cv - base_rows, ct)
                pltpu.async_copy(tsc_hbm.at[ridx_v], msg_v, sem)

            def _rmw(cidx_v, msg_v):
                for g in range(bsz // _LANES):
                    cvec = cidx_v[pl.ds(g * _LANES, _LANES)]
                    rs = [cvec[j] for j in range(_LANES)]

                    @plsc.parallel_loop(0, d // _LANES, 1, unroll=4)
                    def _qbody(q, g=g, rs=rs):
                        for j in range(_LANES):
                            plsc.addupdate(
                                acc_v.at[rs[j], pl.ds(q * _LANES, _LANES)],
                                msg_v[g * _LANES + j,
                                      pl.ds(q * _LANES, _LANES)])

            @pl.when(nb > 0)
            def _prologue():
                _load_mask_start(0, ridx_a, cidx_a, msg_a, sem_a)

            def _pair(it, carry):
                b0 = it * 2
                b1 = b0 + 1
                pltpu.make_async_copy(tsc_hbm.at[ridx_a], msg_a,
                                      sem_a).wait()

                @pl.when(b1 < nb)
                def _startb():
                    _load_mask_start(b1, ridx_b, cidx_b, msg_b, sem_b)

                _rmw(cidx_a, msg_a)

                @pl.when(b1 < nb)
                def _dob():
                    pltpu.make_async_copy(tsc_hbm.at[ridx_b], msg_b,
                                          sem_b).wait()

                    @pl.when(b0 + 2 < nb)
                    def _starta():
                        _load_mask_start(b0 + 2, ridx_a, cidx_a, msg_a,
                                         sem_a)

                    _rmw(cidx_b, msg_b)

                return carry

            lax.fori_loop(0, (nb + 1) // 2, _pair, 0)
            pltpu.sync_copy(acc_v.at[pl.ds(0, ct)],
                            out_hbm.at[pl.ds(base_rows, ct)])
            return carry0

        lax.fori_loop(0, sweeps, _sweep, 0)

    return propagate_kernel


def _bin_weights_body(w1_ref, w2_ref, w3_ref, s1_ref, s2_ref, s3_ref, m1_ref,
                      m2_ref, m3_ref):
    for wr, sr, mr in ((w1_ref, s1_ref, m1_ref), (w2_ref, s2_ref, m2_ref),
                       (w3_ref, s3_ref, m3_ref)):
        w = wr[...]
        mr[...] = jnp.mean(jnp.abs(w), axis=1)[None, :]
        sr[...] = jnp.sign(w).astype(jnp.bfloat16)


def _binarize_weights(W1, W2, W3):
    h, din = W1.shape
    out_d = W3.shape[0]
    return pl.pallas_call(
        _bin_weights_body,
        out_shape=[
            jax.ShapeDtypeStruct((h, din), jnp.bfloat16),
            jax.ShapeDtypeStruct((h, h), jnp.bfloat16),
            jax.ShapeDtypeStruct((out_d, h), jnp.bfloat16),
            jax.ShapeDtypeStruct((1, h), jnp.float32),
            jax.ShapeDtypeStruct((1, h), jnp.float32),
            jax.ShapeDtypeStruct((1, out_d), jnp.float32),
        ],
    )(W1, W2, W3)


def _dinv_of(deg_block):
    deg = jnp.sum(deg_block, axis=1) + 1.0
    return lax.rsqrt(deg)[:, None]


def _binlinear(h, ws_ref, wm_ref, b_ref, dinv):
    alpha = jnp.mean(jnp.abs(h), axis=1, keepdims=True)
    hs = jnp.sign(h).astype(jnp.bfloat16)
    acc = lax.dot_general(hs, ws_ref[...], (((1,), (1,)), ((), ())),
                          preferred_element_type=jnp.float32)
    return dinv * (acc * (alpha * wm_ref[...]) + b_ref[...])


def _d_pre_body(x_ref, deg_ref, out_ref):
    inv0 = (1.0 + _EPS) ** -0.5
    h = x_ref[...] * inv0
    dinv = _dinv_of(deg_ref[...])
    alpha = jnp.mean(jnp.abs(h), axis=1, keepdims=True)
    val = dinv * (alpha * jnp.sign(h))
    pad = jnp.zeros((val.shape[0], _DE - val.shape[1] - 1), jnp.float32)
    out_ref[...] = jnp.concatenate([val, dinv, pad], axis=1)


def _d_first2_body(agg_ref, pre_ref, deg_ref, ws1_ref, wm1_ref, b1_ref,
                   ws2_ref, wm2_ref, b2_ref, out_ref):
    din = ws1_ref.shape[1]
    dinv = _dinv_of(deg_ref[...])
    pe = dinv * (agg_ref[...] + pre_ref[...])
    p1 = pe[:, :din]
    sv = pe[:, din][:, None]
    t1 = lax.dot_general(p1, ws1_ref[...].astype(jnp.float32),
                         (((1,), (1,)), ((), ())),
                         preferred_element_type=jnp.float32)
    h1 = t1 * wm1_ref[...] + sv * b1_ref[...]
    out_ref[...] = _binlinear(h1, ws2_ref, wm2_ref, b2_ref, dinv)


def _d_mid_body(agg_ref, tsc_ref, deg_ref, ws_ref, wm_ref, b_ref, out_ref):
    dinv = _dinv_of(deg_ref[...])
    h = dinv * (agg_ref[...] + tsc_ref[...])
    out_ref[...] = _binlinear(h, ws_ref, wm_ref, b_ref, dinv)


def _d_final_body(agg_ref, tsc_ref, deg_ref, out_ref):
    dinv = _dinv_of(deg_ref[...])
    h = dinv * (agg_ref[...] + tsc_ref[...])
    mx = jnp.max(h, axis=1, keepdims=True)
    lse = jnp.log(jnp.sum(jnp.exp(h - mx), axis=1, keepdims=True)) + mx
    out_ref[...] = h - lse


def _row_spec(din):
    return pl.BlockSpec((_RB, din), lambda i: (i, 0))


def _deg_spec(n1):
    return pl.BlockSpec((_RB, _NW), lambda i: (i, 0))


def _full_spec(shape):
    return pl.BlockSpec(shape, lambda i: tuple(0 for _ in shape))


def _dense_pre(x, deg16):
    n, din = x.shape
    return pl.pallas_call(
        _d_pre_body,
        grid=(n // _RB,),
        in_specs=[_row_spec(din), _deg_spec(n + 1)],
        out_specs=_row_spec(_DE),
        out_shape=jax.ShapeDtypeStruct((n, _DE), jnp.float32),
    )(x, deg16)


def _dense_first2(agg, pre, deg16, ws1, wm1, b1r, ws2, wm2, b2r):
    n = pre.shape[0]
    dout = ws2.shape[0]
    return pl.pallas_call(
        _d_first2_body,
        grid=(n // _RB,),
        in_specs=[
            _row_spec(_DE),
            _row_spec(_DE),
            _deg_spec(n + 1),
            _full_spec(ws1.shape),
            _full_spec(wm1.shape),
            _full_spec(b1r.shape),
            _full_spec(ws2.shape),
            _full_spec(wm2.shape),
            _full_spec(b2r.shape),
        ],
        out_specs=_row_spec(dout),
        out_shape=jax.ShapeDtypeStruct((n, dout), jnp.float32),
    )(agg, pre, deg16, ws1, wm1, b1r, ws2, wm2, b2r)


def _dense_mid(agg, tsc, deg16, ws, wm, b2d):
    n, din = tsc.shape
    dout = ws.shape[0]
    return pl.pallas_call(
        _d_mid_body,
        grid=(n // _RB,),
        in_specs=[
            _row_spec(din),
            _row_spec(din),
            _deg_spec(n + 1),
            _full_spec(ws.shape),
            _full_spec(wm.shape),
            _full_spec(b2d.shape),
        ],
        out_specs=_row_spec(dout),
        out_shape=jax.ShapeDtypeStruct((n, dout), jnp.float32),
    )(agg, tsc, deg16, ws, wm, b2d)


def _dense_final(agg, tsc, deg16):
    n, d = tsc.shape
    return pl.pallas_call(
        _d_final_body,
        grid=(n // _RB,),
        in_specs=[_row_spec(d), _row_spec(d), _deg_spec(n + 1)],
        out_specs=_row_spec(d),
        out_shape=jax.ShapeDtypeStruct((n, d), jnp.float32),
    )(agg, tsc, deg16)


def kernel(x, W1, b1, W2, b2, W3, b3, edge_index):
    n, _ = x.shape
    h1 = W1.shape[0]
    out_d = W3.shape[0]
    e = edge_index.shape[1]

    row = edge_index[0]
    col = edge_index[1]
    order = jnp.argsort(col)
    col_s = jnp.take(col, order)
    row_s = jnp.take(row, order)
    nb_deg = -(-e // _BD)
    e_pad = (-(-nb_deg // _NW)) * _NW * _BD
    pad = jnp.zeros((e_pad - e,), jnp.int32)
    row_p = jnp.concatenate([row_s, pad])
    col_p = jnp.concatenate([col_s, pad])

    def _bounds_for(d):
        ct = _PROP_CFG[d][0]
        nchunks = _ND // ct
        starts = jnp.searchsorted(
            col_s, jnp.arange(0, _ND + ct, ct, dtype=jnp.int32)).astype(
                jnp.int32)
        bnd = jnp.stack([starts[:-1], starts[1:]], axis=1)
        bnd = jnp.concatenate(
            [bnd, jnp.zeros((nchunks, 14), jnp.int32)], axis=1)
        return bnd.reshape(nchunks, 1, 16)

    zeros_deg = jnp.zeros((_ND // _LANES, _LANES), jnp.float32)
    zrows_h = jnp.zeros((_PROP_CFG[h1][0], h1), jnp.float32)
    zrows_o = jnp.zeros((_PROP_CFG[out_d][0], out_d), jnp.float32)

    deg_parts = _make_degree_kernel(e, e_pad, nb_deg)(row_p, zeros_deg)
    deg_flat = deg_parts.reshape(_NW, _ND).T
    ws1, ws2, ws3, m1, m2, m3 = _binarize_weights(W1, W2, W3)
    b1r, b2r, b3r = b1[None, :], b2[None, :], b3[None, :]

    prop_e = _make_propagate_kernel(n, _DE)
    prop_h = _make_propagate_kernel(n, h1)
    prop_o = _make_propagate_kernel(n, out_d)
    bnd_e = _bounds_for(_DE)
    bnd_h = _bounds_for(h1)
    bnd_o = _bounds_for(out_d)
    zrows_e = jnp.zeros((_PROP_CFG[_DE][0], _DE), jnp.float32)

    pre0 = _dense_pre(x, deg_flat)
    a0 = prop_e(pre0, row_p, col_p, bnd_e, zrows_e)
    t2 = _dense_first2(a0, pre0, deg_flat, ws1, m1, b1r, ws2, m2, b2r)
    a2 = prop_h(t2, row_p, col_p, bnd_h, zrows_h)
    t3 = _dense_mid(a2, t2, deg_flat, ws3, m3, b3r)
    a3 = prop_o(t3, row_p, col_p, bnd_o, zrows_o)
    return _dense_final(a3, t3, deg_flat)
